# baseline (speedup 1.0000x reference)
import functools

import jax
import jax.numpy as jnp
from jax import lax
from jax.experimental import pallas as pl
from jax.experimental.pallas import tpu as pltpu
from jax.experimental.pallas import tpu_sc as plsc

N_ITEMS = 1000000
K = 32
BATCH = 16384

_info = plsc.get_sparse_core_info()
_NC = _info.num_cores
_NS = _info.num_subcores
_L = _info.num_lanes
_NW = _NC * _NS
_BPW = BATCH // _NW

_mesh = plsc.VectorSubcoreMesh(core_axis_name="c", subcore_axis_name="s")


@functools.partial(
    pl.kernel,
    mesh=_mesh,
    out_type=jax.ShapeDtypeStruct((BATCH,), jnp.float32),
    scratch_types=[
        pltpu.VMEM((_BPW,), jnp.int32),
        pltpu.VMEM((_BPW,), jnp.int32),
        pltpu.VMEM((_BPW, K), jnp.float32),
        pltpu.VMEM((_BPW, K), jnp.float32),
        pltpu.VMEM((_BPW,), jnp.float32),
        pltpu.SemaphoreType.DMA,
        pltpu.SemaphoreType.DMA,
    ],
    compiler_params=pltpu.CompilerParams(
        use_tc_tiling_on_sc=False, needs_layout_passes=False
    ),
)
def _mf_dot(u_idx_hbm, v_idx_hbm, u_hbm, v_hbm, out_hbm,
            uidx_v, vidx_v, urows_v, vrows_v, out_v, sem_u, sem_v):
    wid = lax.axis_index("s") * _NC + lax.axis_index("c")
    base = wid * _BPW

    pltpu.sync_copy(u_idx_hbm.at[pl.ds(base, _BPW)], uidx_v)
    pltpu.sync_copy(v_idx_hbm.at[pl.ds(base, _BPW)], vidx_v)

    cp_u = pltpu.async_copy(u_hbm.at[uidx_v], urows_v, sem_u)
    cp_v = pltpu.async_copy(v_hbm.at[vidx_v], vrows_v, sem_v)
    cp_u.wait()
    cp_v.wait()

    def group(g, carry):
        rows = g * _L + lax.iota(jnp.int32, _L)
        acc = jnp.zeros((_L,), jnp.float32)
        for j in range(K):
            cols = jnp.full((_L,), j, jnp.int32)
            a = plsc.load_gather(urows_v, [rows, cols])
            b = plsc.load_gather(vrows_v, [rows, cols])
            acc = acc + a * b
        out_v[pl.ds(g * _L, _L)] = acc
        return carry

    lax.fori_loop(0, _BPW // _L, group, 0)

    pltpu.sync_copy(out_v, out_hbm.at[pl.ds(base, _BPW)])


def kernel(u_idx, v_idx, U, V):
    return _mf_dot(u_idx.astype(jnp.int32), v_idx.astype(jnp.int32), U, V)
